# baseline (speedup 1.0000x reference)
import functools

import jax
import jax.numpy as jnp
from jax import lax
from jax.experimental import pallas as pl
from jax.experimental.pallas import tpu as pltpu
from jax.experimental.pallas import tpu_sc as plsc

B, L, V, D = 1024, 200, 1000000, 64
NC, NS = 2, 16
NW = NC * NS
BPW = B // NW
NCOMB = 3 * L
NWIN = L // 16


@functools.partial(
    pl.kernel,
    out_type=jax.ShapeDtypeStruct((B, L, D), jnp.float32),
    mesh=plsc.VectorSubcoreMesh(core_axis_name="c", subcore_axis_name="s"),
    scratch_types=[
        pltpu.VMEM((BPW, L), jnp.int32),
        pltpu.VMEM((BPW, L), jnp.int32),
        pltpu.VMEM((3, D), jnp.float32),
        pltpu.VMEM((NCOMB * D,), jnp.float32),
        pltpu.VMEM((L, D), jnp.float32),
        pltpu.VMEM((L, D), jnp.float32),
        pltpu.SMEM((L,), jnp.int32),
        pltpu.SMEM((L,), jnp.int32),
        pltpu.SemaphoreType.DMA,
        pltpu.SemaphoreType.DMA,
        pltpu.SemaphoreType.DMA,
        pltpu.SemaphoreType.DMA,
    ],
)
def _sc_embed(x_h, seg_h, table_h, segt_h, pos_h, out_h,
              xv, sv, segt_v, comb_v, buf0, buf1, sm0, sm1,
              gsem0, gsem1, osem0, osem1):
    cid = lax.axis_index("c")
    sid = lax.axis_index("s")
    wid = sid * NC + cid
    bbase = pl.multiple_of(wid * BPW, BPW)

    pltpu.sync_copy(x_h.at[pl.ds(bbase, BPW), :], xv)
    pltpu.sync_copy(seg_h.at[pl.ds(bbase, BPW), :], sv)
    pltpu.sync_copy(segt_h, segt_v)
    pltpu.sync_copy(pos_h.at[pl.ds(0, L), :], buf0)

    def comb_body(l, carry):
        cbase = pl.multiple_of(3 * l * D, D)
        for s in range(3):
            for q in range(D // 16):
                pv = buf0[l, pl.ds(q * 16, 16)]
                tv = segt_v[s, pl.ds(q * 16, 16)]
                comb_v[pl.ds(cbase + s * D + q * 16, 16)] = pv + tv
        return carry

    lax.fori_loop(0, L, comb_body, 0)

    bufs = (buf0, buf1)
    stash = (sm0, sm1)
    gsems = (gsem0, gsem1)
    osems = (osem0, osem1)

    def fetch(bb, slot):
        buf, sm = bufs[slot], stash[slot]

        def win_body(w, carry):
            wstart = pl.multiple_of(w * 16, 16)
            xvec = xv[bb, pl.ds(wstart, 16)]
            svec = sv[bb, pl.ds(wstart, 16)]
            pvec = xvec * 4 + svec
            for i in range(16):
                p = pvec[i]
                pltpu.async_copy(table_h.at[p >> 2], buf.at[wstart + i],
                                 gsems[slot])
                sm[wstart + i] = p & 3
            return carry

        lax.fori_loop(0, NWIN, win_body, 0)
        xvec = xv[bb, pl.ds(L - 16, 16)]
        svec = sv[bb, pl.ds(L - 16, 16)]
        pvec = xvec * 4 + svec
        for i in range(8, 16):
            p = pvec[i]
            pltpu.async_copy(table_h.at[p >> 2], buf.at[L - 16 + i],
                             gsems[slot])
            sm[L - 16 + i] = p & 3

    def process(bb, slot):
        buf, sm = bufs[slot], stash[slot]
        pltpu.make_async_copy(out_h.at[0], buf, gsems[slot]).wait()

        def add_body(j, carry):
            aoff = pl.multiple_of((3 * j + sm[j]) * D, D)
            for q in range(D // 16):
                av = comb_v[pl.ds(aoff + q * 16, 16)]
                buf[j, pl.ds(q * 16, 16)] += av
            return carry

        lax.fori_loop(0, L, add_body, 0)
        pltpu.async_copy(buf, out_h.at[bbase + bb], osems[slot])

    def drain_out(slot):
        pltpu.make_async_copy(bufs[slot], out_h.at[0], osems[slot]).wait()

    fetch(0, 0)

    def pair_body(h, carry):
        bb = h * 2

        @pl.when(h > 0)
        def _():
            drain_out(1)
        fetch(bb + 1, 1)
        process(bb, 0)

        drain_out(0)

        @pl.when(h + 1 < BPW // 2)
        def _():
            fetch(bb + 2, 0)
        process(bb + 1, 1)
        return carry

    lax.fori_loop(0, BPW // 2, pair_body, 0)
    drain_out(1)


def kernel(x, segment_info, table, seg_table, pos):
    return _sc_embed(x.astype(jnp.int32), segment_info.astype(jnp.int32),
                     table, seg_table, pos)


# speedup vs baseline: 1.0479x; 1.0479x over previous
import functools

import jax
import jax.numpy as jnp
from jax import lax
from jax.experimental import pallas as pl
from jax.experimental.pallas import tpu as pltpu
from jax.experimental.pallas import tpu_sc as plsc

B, L, V, D = 1024, 200, 1000000, 64
NC, NS = 2, 16
NW = NC * NS
BPW = B // NW
NCOMB = 3 * L
NWIN = L // 16


@functools.partial(
    pl.kernel,
    out_type=jax.ShapeDtypeStruct((B, L, D), jnp.float32),
    mesh=plsc.VectorSubcoreMesh(core_axis_name="c", subcore_axis_name="s"),
    scratch_types=[
        pltpu.VMEM((BPW, L), jnp.int32),
        pltpu.VMEM((BPW, L), jnp.int32),
        pltpu.VMEM((3, D), jnp.float32),
        pltpu.VMEM((NCOMB * D,), jnp.float32),
        pltpu.VMEM((L, D), jnp.float32),
        pltpu.VMEM((L, D), jnp.float32),
        pltpu.SMEM((L,), jnp.int32),
        pltpu.SMEM((L,), jnp.int32),
        pltpu.SemaphoreType.DMA,
        pltpu.SemaphoreType.DMA,
        pltpu.SemaphoreType.DMA,
        pltpu.SemaphoreType.DMA,
    ],
)
def _sc_embed(x_h, seg_h, table_h, segt_h, pos_h, out_h,
              xv, sv, segt_v, comb_v, buf0, buf1, sm0, sm1,
              gsem0, gsem1, osem0, osem1):
    cid = lax.axis_index("c")
    sid = lax.axis_index("s")
    wid = sid * NC + cid
    bbase = pl.multiple_of(wid * BPW, BPW)

    pltpu.sync_copy(x_h.at[pl.ds(bbase, BPW), :], xv)
    pltpu.sync_copy(seg_h.at[pl.ds(bbase, BPW), :], sv)
    pltpu.sync_copy(segt_h, segt_v)
    pltpu.sync_copy(pos_h.at[pl.ds(0, L), :], buf0)

    def comb_body(l, carry):
        cbase = pl.multiple_of(3 * l * D, D)
        for s in range(3):
            for q in range(D // 16):
                pv = buf0[l, pl.ds(q * 16, 16)]
                tv = segt_v[s, pl.ds(q * 16, 16)]
                comb_v[pl.ds(cbase + s * D + q * 16, 16)] = pv + tv
        return carry

    lax.fori_loop(0, L, comb_body, 0)

    bufs = (buf0, buf1)
    stash = (sm0, sm1)
    gsems = (gsem0, gsem1)
    osems = (osem0, osem1)

    def fetch(bb, slot):
        buf, sm = bufs[slot], stash[slot]

        def win_body(w, carry):
            wstart = pl.multiple_of(w * 16, 16)
            xvec = xv[bb, pl.ds(wstart, 16)]
            svec = sv[bb, pl.ds(wstart, 16)]
            pvec = xvec * 4 + svec
            for i in range(16):
                p = pvec[i]
                pltpu.async_copy(table_h.at[p >> 2], buf.at[wstart + i],
                                 gsems[slot])
                sm[wstart + i] = p & 3
            return carry

        lax.fori_loop(0, NWIN, win_body, 0)
        xvec = xv[bb, pl.ds(L - 16, 16)]
        svec = sv[bb, pl.ds(L - 16, 16)]
        pvec = xvec * 4 + svec
        for i in range(8, 16):
            p = pvec[i]
            pltpu.async_copy(table_h.at[p >> 2], buf.at[L - 16 + i],
                             gsems[slot])
            sm[L - 16 + i] = p & 3

    def process(bb, slot):
        buf, sm = bufs[slot], stash[slot]
        pltpu.make_async_copy(out_h.at[0], buf, gsems[slot]).wait()

        def add_body(jj, carry):
            for u in range(2):
                j = jj * 2 + u
                aoff = pl.multiple_of((3 * j + sm[j]) * D, D)
                for q in range(D // 16):
                    av = comb_v[pl.ds(aoff + q * 16, 16)]
                    plsc.addupdate(buf.at[j, pl.ds(q * 16, 16)], av)
            return carry

        lax.fori_loop(0, L // 2, add_body, 0)
        pltpu.async_copy(buf, out_h.at[bbase + bb], osems[slot])

    def drain_out(slot):
        pltpu.make_async_copy(bufs[slot], out_h.at[0], osems[slot]).wait()

    fetch(0, 0)

    def pair_body(h, carry):
        bb = h * 2

        @pl.when(h > 0)
        def _():
            drain_out(1)
        fetch(bb + 1, 1)
        process(bb, 0)

        drain_out(0)

        @pl.when(h + 1 < BPW // 2)
        def _():
            fetch(bb + 2, 0)
        process(bb + 1, 1)
        return carry

    lax.fori_loop(0, BPW // 2, pair_body, 0)
    drain_out(1)


def kernel(x, segment_info, table, seg_table, pos):
    return _sc_embed(x.astype(jnp.int32), segment_info.astype(jnp.int32),
                     table, seg_table, pos)


# speedup vs baseline: 1.0880x; 1.0383x over previous
import functools

import jax
import jax.numpy as jnp
from jax import lax
from jax.experimental import pallas as pl
from jax.experimental.pallas import tpu as pltpu
from jax.experimental.pallas import tpu_sc as plsc

B, L, V, D = 1024, 200, 1000000, 64
NC, NS = 2, 16
NW = NC * NS
BPW = B // NW
NCOMB = 3 * L
NWIN = L // 16


@functools.partial(
    pl.kernel,
    out_type=jax.ShapeDtypeStruct((B, L, D), jnp.float32),
    mesh=plsc.VectorSubcoreMesh(core_axis_name="c", subcore_axis_name="s"),
    scratch_types=[
        pltpu.VMEM((BPW, L), jnp.int32),
        pltpu.VMEM((BPW, L), jnp.int32),
        pltpu.VMEM((3, D), jnp.float32),
        pltpu.VMEM((NCOMB * D,), jnp.float32),
        pltpu.VMEM((L, D), jnp.float32),
        pltpu.VMEM((L, D), jnp.float32),
        pltpu.SMEM((L,), jnp.int32),
        pltpu.SMEM((L,), jnp.int32),
        pltpu.SemaphoreType.DMA,
        pltpu.SemaphoreType.DMA,
        pltpu.SemaphoreType.DMA,
        pltpu.SemaphoreType.DMA,
    ],
)
def _sc_embed(x_h, seg_h, table_h, segt_h, pos_h, out_h,
              xv, sv, segt_v, comb_v, buf0, buf1, sm0, sm1,
              gsem0, gsem1, osem0, osem1):
    cid = lax.axis_index("c")
    sid = lax.axis_index("s")
    wid = sid * NC + cid
    bbase = pl.multiple_of(wid * BPW, BPW)

    pltpu.sync_copy(x_h.at[pl.ds(bbase, BPW), :], xv)
    pltpu.sync_copy(seg_h.at[pl.ds(bbase, BPW), :], sv)
    pltpu.sync_copy(segt_h, segt_v)
    pltpu.sync_copy(pos_h.at[pl.ds(0, L), :], buf0)

    def comb_body(l, carry):
        cbase = pl.multiple_of(3 * l * D, D)
        for s in range(3):
            for q in range(D // 16):
                pv = buf0[l, pl.ds(q * 16, 16)]
                tv = segt_v[s, pl.ds(q * 16, 16)]
                comb_v[pl.ds(cbase + s * D + q * 16, 16)] = pv + tv
        return carry

    lax.fori_loop(0, L, comb_body, 0)

    bufs = (buf0, buf1)
    stash = (sm0, sm1)
    gsems = (gsem0, gsem1)
    osems = (osem0, osem1)

    def fetch(bb, slot):
        buf, sm = bufs[slot], stash[slot]

        def win_body(w, carry):
            for u in range(2):
                wstart = pl.multiple_of((w * 2 + u) * 16, 16)
                xvec = xv[bb, pl.ds(wstart, 16)]
                svec = sv[bb, pl.ds(wstart, 16)]
                pvec = xvec * 4 + svec
                for i in range(16):
                    p = pvec[i]
                    pltpu.async_copy(table_h.at[p >> 2], buf.at[wstart + i],
                                     gsems[slot])
                    sm[wstart + i] = p & 3
            return carry

        lax.fori_loop(0, NWIN // 2, win_body, 0)
        xvec = xv[bb, pl.ds(L - 16, 16)]
        svec = sv[bb, pl.ds(L - 16, 16)]
        pvec = xvec * 4 + svec
        for i in range(8, 16):
            p = pvec[i]
            pltpu.async_copy(table_h.at[p >> 2], buf.at[L - 16 + i],
                             gsems[slot])
            sm[L - 16 + i] = p & 3

    def process(bb, slot):
        buf, sm = bufs[slot], stash[slot]
        pltpu.make_async_copy(out_h.at[0], buf, gsems[slot]).wait()

        def add_body(jj, carry):
            for u in range(2):
                j = jj * 2 + u
                aoff = pl.multiple_of((3 * j + sm[j]) * D, D)
                for q in range(D // 16):
                    av = comb_v[pl.ds(aoff + q * 16, 16)]
                    plsc.addupdate(buf.at[j, pl.ds(q * 16, 16)], av)
            return carry

        lax.fori_loop(0, L // 2, add_body, 0)
        pltpu.async_copy(buf, out_h.at[bbase + bb], osems[slot])

    def drain_out(slot):
        pltpu.make_async_copy(bufs[slot], out_h.at[0], osems[slot]).wait()

    fetch(0, 0)

    def pair_body(h, carry):
        bb = h * 2

        @pl.when(h > 0)
        def _():
            drain_out(1)
        fetch(bb + 1, 1)
        process(bb, 0)
        process(bb + 1, 1)

        drain_out(0)

        @pl.when(h + 1 < BPW // 2)
        def _():
            fetch(bb + 2, 0)
        return carry

    lax.fori_loop(0, BPW // 2, pair_body, 0)
    drain_out(1)


def kernel(x, segment_info, table, seg_table, pos):
    return _sc_embed(x.astype(jnp.int32), segment_info.astype(jnp.int32),
                     table, seg_table, pos)


# speedup vs baseline: 1.1340x; 1.0423x over previous
import functools

import jax
import jax.numpy as jnp
from jax import lax
from jax.experimental import pallas as pl
from jax.experimental.pallas import tpu as pltpu
from jax.experimental.pallas import tpu_sc as plsc

B, L, V, D = 1024, 200, 1000000, 64
NC, NS = 2, 16
NW = NC * NS
BPW = B // NW
NCOMB = 3 * L
NWIN = L // 16


@functools.partial(
    pl.kernel,
    out_type=jax.ShapeDtypeStruct((B * L, D), jnp.float32),
    mesh=plsc.VectorSubcoreMesh(core_axis_name="c", subcore_axis_name="s"),
    scratch_types=[
        pltpu.VMEM((BPW, L), jnp.int32),
        pltpu.VMEM((BPW, L), jnp.int32),
        pltpu.VMEM((3, D), jnp.float32),
        pltpu.VMEM((NCOMB * D,), jnp.float32),
        pltpu.VMEM((L, D), jnp.float32),
        pltpu.VMEM((L, D), jnp.float32),
        pltpu.SMEM((L,), jnp.int32),
        pltpu.SMEM((L,), jnp.int32),
        pltpu.SemaphoreType.DMA,
        pltpu.SemaphoreType.DMA,
        pltpu.SemaphoreType.DMA,
        pltpu.SemaphoreType.DMA,
    ],
)
def _sc_embed(x_h, seg_h, table_h, segt_h, pos_h, out_h,
              xv, sv, segt_v, comb_v, buf0, buf1, sm0, sm1,
              gsem0, gsem1, osem0, osem1):
    cid = lax.axis_index("c")
    sid = lax.axis_index("s")
    wid = sid * NC + cid
    bbase = pl.multiple_of(wid * BPW, BPW)

    pltpu.sync_copy(x_h.at[pl.ds(bbase, BPW), :], xv)
    pltpu.sync_copy(seg_h.at[pl.ds(bbase, BPW), :], sv)
    pltpu.sync_copy(segt_h, segt_v)
    pltpu.sync_copy(pos_h.at[pl.ds(0, L), :], buf0)

    def comb_body(l, carry):
        cbase = pl.multiple_of(3 * l * D, D)
        for s in range(3):
            for q in range(D // 16):
                pv = buf0[l, pl.ds(q * 16, 16)]
                tv = segt_v[s, pl.ds(q * 16, 16)]
                comb_v[pl.ds(cbase + s * D + q * 16, 16)] = pv + tv
        return carry

    lax.fori_loop(0, L, comb_body, 0)

    bufs = (buf0, buf1)
    stash = (sm0, sm1)
    gsems = (gsem0, gsem1)
    osems = (osem0, osem1)

    def fetch(bb, slot):
        buf, sm = bufs[slot], stash[slot]

        def win_body(w, carry):
            for u in range(2):
                wstart = pl.multiple_of((w * 2 + u) * 16, 16)
                xvec = xv[bb, pl.ds(wstart, 16)]
                svec = sv[bb, pl.ds(wstart, 16)]
                pvec = xvec * 4 + svec
                for i in range(16):
                    p = pvec[i]
                    pltpu.async_copy(table_h.at[p >> 2], buf.at[wstart + i],
                                     gsems[slot])
                    sm[wstart + i] = p & 3
            return carry

        lax.fori_loop(0, NWIN // 2, win_body, 0)
        xvec = xv[bb, pl.ds(L - 16, 16)]
        svec = sv[bb, pl.ds(L - 16, 16)]
        pvec = xvec * 4 + svec
        for i in range(8, 16):
            p = pvec[i]
            pltpu.async_copy(table_h.at[p >> 2], buf.at[L - 16 + i],
                             gsems[slot])
            sm[L - 16 + i] = p & 3

    def process(bb, slot):
        buf, sm = bufs[slot], stash[slot]
        pltpu.make_async_copy(out_h.at[pl.ds(0, L), :], buf, gsems[slot]).wait()

        def add_body(jj, carry):
            for u in range(4):
                j = jj * 4 + u
                aoff = pl.multiple_of((3 * j + sm[j]) * D, D)
                for q in range(D // 16):
                    av = comb_v[pl.ds(aoff + q * 16, 16)]
                    plsc.addupdate(buf.at[j, pl.ds(q * 16, 16)], av)
            return carry

        lax.fori_loop(0, L // 4, add_body, 0)
        pltpu.async_copy(buf, out_h.at[pl.ds((bbase + bb) * L, L), :], osems[slot])

    def drain_out(slot):
        pltpu.make_async_copy(bufs[slot], out_h.at[pl.ds(0, L), :], osems[slot]).wait()

    fetch(0, 0)

    def pair_body(h, carry):
        bb = h * 2

        @pl.when(h > 0)
        def _():
            drain_out(1)
        fetch(bb + 1, 1)
        process(bb, 0)
        process(bb + 1, 1)

        drain_out(0)

        @pl.when(h + 1 < BPW // 2)
        def _():
            fetch(bb + 2, 0)
        return carry

    lax.fori_loop(0, BPW // 2, pair_body, 0)
    drain_out(1)


def kernel(x, segment_info, table, seg_table, pos):
    out = _sc_embed(x.astype(jnp.int32), segment_info.astype(jnp.int32),
                    table, seg_table, pos)
    return out.reshape(B, L, D)
